# baseline (speedup 1.0000x reference)
import functools

import jax
import jax.numpy as jnp
from jax import lax
from jax.experimental import pallas as pl
from jax.experimental.pallas import tpu as pltpu
from jax.experimental.pallas import tpu_sc as plsc

_CHUNK = 8
_HI0, _HIN = 96, 32
_BB = 4


def _sc_hi_body(x_hbm, hi_hbm, buf0, buf1, si0, si1, so0, so1):
    c = lax.axis_index("c")
    s = lax.axis_index("s")
    w = s * 2 + c
    bufs = (buf0, buf1)
    in_sems = (si0, si1)
    out_sems = (so0, so1)

    tasks = []
    for lb in range(2):
        for k in range(_HIN // _CHUNK):
            tasks.append((lb, k * _CHUNK))

    def make_in(i):
        lb, d0 = tasks[i]
        return pltpu.make_async_copy(
            x_hbm.at[w * 2 + lb, pl.ds(_HI0 + d0, _CHUNK)], bufs[i % 2], in_sems[i % 2]
        )

    def make_out(i):
        lb, d0 = tasks[i]
        return pltpu.make_async_copy(
            bufs[i % 2], hi_hbm.at[w * 2 + lb, pl.ds(d0, _CHUNK)], out_sems[i % 2]
        )

    n = len(tasks)
    make_in(0).start()
    make_in(1).start()
    for i in range(n):
        make_in(i).wait()
        make_out(i).start()
        make_out(i).wait()
        if i + 2 < n:
            make_in(i + 2).start()


def _tc_lomid_body(x_ref, lo_ref, mid_ref):
    lo_ref[...] = x_ref[:, 0:48, :]
    mid_ref[...] = x_ref[:, 48:96, :]


def kernel(x, idx_low, idx_mid, idx_high):
    B, _, R, C = x.shape
    x3 = x.reshape(B, R, C)

    mesh = plsc.VectorSubcoreMesh(core_axis_name="c", subcore_axis_name="s")
    sc_hi = functools.partial(
        pl.kernel,
        mesh=mesh,
        out_type=jax.ShapeDtypeStruct((B, _HIN, C), x.dtype),
        scratch_types=[
            pltpu.VMEM((_CHUNK, C), x.dtype),
            pltpu.VMEM((_CHUNK, C), x.dtype),
            pltpu.SemaphoreType.DMA,
            pltpu.SemaphoreType.DMA,
            pltpu.SemaphoreType.DMA,
            pltpu.SemaphoreType.DMA,
        ],
    )(_sc_hi_body)
    hi = sc_hi(x3)

    lo, mid = pl.pallas_call(
        _tc_lomid_body,
        grid=(B // _BB,),
        in_specs=[pl.BlockSpec((_BB, 96, C), lambda b: (b, 0, 0))],
        out_specs=(
            pl.BlockSpec((_BB, 48, C), lambda b: (b, 0, 0)),
            pl.BlockSpec((_BB, 48, C), lambda b: (b, 0, 0)),
        ),
        out_shape=(
            jax.ShapeDtypeStruct((B, 48, C), x.dtype),
            jax.ShapeDtypeStruct((B, 48, C), x.dtype),
        ),
    )(x3)

    return lo.reshape(B, 1, 48, C), mid, hi.reshape(B, 1, 32, C)


# speedup vs baseline: 1.0157x; 1.0157x over previous
import functools

import jax
import jax.numpy as jnp
from jax import lax
from jax.experimental import pallas as pl
from jax.experimental.pallas import tpu as pltpu
from jax.experimental.pallas import tpu_sc as plsc

_CHUNK = 8
_HI0, _HIN = 96, 32
_BB = 8


def _sc_hi_body(x_hbm, hi_hbm, buf0, buf1, si0, si1, so0, so1):
    c = lax.axis_index("c")
    s = lax.axis_index("s")
    w = s * 2 + c
    bufs = (buf0, buf1)
    in_sems = (si0, si1)
    out_sems = (so0, so1)

    tasks = []
    for lb in range(2):
        for k in range(_HIN // _CHUNK):
            tasks.append((lb, k * _CHUNK))

    def make_in(i):
        lb, d0 = tasks[i]
        return pltpu.make_async_copy(
            x_hbm.at[w * 2 + lb, pl.ds(_HI0 + d0, _CHUNK)], bufs[i % 2], in_sems[i % 2]
        )

    def make_out(i):
        lb, d0 = tasks[i]
        return pltpu.make_async_copy(
            bufs[i % 2], hi_hbm.at[w * 2 + lb, pl.ds(d0, _CHUNK)], out_sems[i % 2]
        )

    n = len(tasks)
    make_in(0).start()
    make_in(1).start()
    for i in range(n):
        make_in(i).wait()
        make_out(i).start()
        make_out(i).wait()
        if i + 2 < n:
            make_in(i + 2).start()


def _tc_lomid_body(x_ref, lo_ref, mid_ref):
    lo_ref[...] = x_ref[:, 0:48, :]
    mid_ref[...] = x_ref[:, 48:96, :]


def kernel(x, idx_low, idx_mid, idx_high):
    B, _, R, C = x.shape
    x3 = x.reshape(B, R, C)

    mesh = plsc.VectorSubcoreMesh(core_axis_name="c", subcore_axis_name="s")
    sc_hi = functools.partial(
        pl.kernel,
        mesh=mesh,
        out_type=jax.ShapeDtypeStruct((B, _HIN, C), x.dtype),
        scratch_types=[
            pltpu.VMEM((_CHUNK, C), x.dtype),
            pltpu.VMEM((_CHUNK, C), x.dtype),
            pltpu.SemaphoreType.DMA,
            pltpu.SemaphoreType.DMA,
            pltpu.SemaphoreType.DMA,
            pltpu.SemaphoreType.DMA,
        ],
    )(_sc_hi_body)
    hi = sc_hi(x3)

    lo, mid = pl.pallas_call(
        _tc_lomid_body,
        grid=(B // _BB,),
        in_specs=[pl.BlockSpec((_BB, 96, C), lambda b: (b, 0, 0))],
        out_specs=(
            pl.BlockSpec((_BB, 48, C), lambda b: (b, 0, 0)),
            pl.BlockSpec((_BB, 48, C), lambda b: (b, 0, 0)),
        ),
        out_shape=(
            jax.ShapeDtypeStruct((B, 48, C), x.dtype),
            jax.ShapeDtypeStruct((B, 48, C), x.dtype),
        ),
    )(x3)

    return lo.reshape(B, 1, 48, C), mid, hi.reshape(B, 1, 32, C)
